# baseline (speedup 1.0000x reference)
import functools

import jax
import jax.numpy as jnp
from jax import lax
from jax.experimental import pallas as pl
from jax.experimental.pallas import tpu as pltpu
from jax.experimental.pallas import tpu_sc as plsc

NUM_ENTITIES = 1000000
NUM_RELATIONS = 1000
EMBED_DIM = 64
BATCH = 16384

_NC = 2
_NS = 16
_NW = _NC * _NS
_BPW = BATCH // _NW
_CHUNK = 128
_NCHUNK = _BPW // _CHUNK
_L = 16
_GROUPS = _BPW // _L


def _rsqrt_nr(s):
    s = jnp.maximum(s, jnp.float32(1e-30))
    i = plsc.bitcast(s, jnp.int32)
    i = jnp.int32(0x5F3759DF) - lax.shift_right_arithmetic(i, jnp.int32(1))
    y = plsc.bitcast(i, jnp.float32)
    half = jnp.float32(0.5)
    three_half = jnp.float32(1.5)
    for _ in range(3):
        y = y * (three_half - half * s * y * y)
    return y


def _transe_body(heads, relations, tails, entity_emb, relation_emb, out,
                 idx_h, idx_r, idx_t, h_rows, r_rows, t_rows, score_v, sem):
    wid = lax.axis_index("s") * _NC + lax.axis_index("c")
    base = wid * _BPW

    for j in range(_NCHUNK):
        off = base + j * _CHUNK
        pltpu.sync_copy(heads.at[pl.ds(off, _CHUNK)], idx_h.at[j])
        pltpu.sync_copy(relations.at[pl.ds(off, _CHUNK)], idx_r.at[j])
        pltpu.sync_copy(tails.at[pl.ds(off, _CHUNK)], idx_t.at[j])

    h2 = h_rows
    r2 = r_rows
    t2 = t_rows
    copies = []
    for j in range(_NCHUNK):
        dst = pl.ds(j * _CHUNK, _CHUNK)
        copies.append(pltpu.async_copy(entity_emb.at[idx_h.at[j]],
                                       h2.at[dst], sem))
        copies.append(pltpu.async_copy(relation_emb.at[idx_r.at[j]],
                                       r2.at[dst], sem))
        copies.append(pltpu.async_copy(entity_emb.at[idx_t.at[j]],
                                       t2.at[dst], sem))
    for c in copies:
        c.wait()

    zero = jnp.zeros((_L,), jnp.float32)
    eps = jnp.float32(1e-12)
    lane = lax.iota(jnp.int32, _L)
    nchunks = EMBED_DIM // _L

    def group(g, carry):
        hh = rr = tt = hr = ht = rt = zero
        for j in range(_L):
            i = g * _L + j
            hh_p = rr_p = tt_p = hr_p = ht_p = rt_p = zero
            for k in range(nchunks):
                sl = pl.ds(k * _L, _L)
                hv = h2[i, sl]
                rv = r2[i, sl]
                tv = t2[i, sl]
                hh_p = hh_p + hv * hv
                rr_p = rr_p + rv * rv
                tt_p = tt_p + tv * tv
                hr_p = hr_p + hv * rv
                ht_p = ht_p + hv * tv
                rt_p = rt_p + rv * tv
            m = lane == j
            hh = jnp.where(m, jnp.sum(hh_p), hh)
            rr = jnp.where(m, jnp.sum(rr_p), rr)
            tt = jnp.where(m, jnp.sum(tt_p), tt)
            hr = jnp.where(m, jnp.sum(hr_p), hr)
            ht = jnp.where(m, jnp.sum(ht_p), ht)
            rt = jnp.where(m, jnp.sum(rt_p), rt)
        a = jnp.float32(1.0) / jnp.maximum(hh * _rsqrt_nr(hh), eps)
        b = jnp.float32(1.0) / jnp.maximum(rr * _rsqrt_nr(rr), eps)
        c = jnp.float32(1.0) / jnp.maximum(tt * _rsqrt_nr(tt), eps)
        s2 = (hh * a * a + rr * b * b + tt * c * c
              + jnp.float32(2.0) * (a * b * hr - a * c * ht - b * c * rt))
        s2 = jnp.maximum(s2, jnp.float32(0.0))
        score_v[pl.ds(g * _L, _L)] = s2 * _rsqrt_nr(s2)
        return carry

    lax.fori_loop(0, _GROUPS, group, 0)

    pltpu.sync_copy(score_v, out.at[pl.ds(base, _BPW)])


@jax.jit
def kernel(heads, relations, tails, entity_emb, relation_emb):
    mesh = plsc.VectorSubcoreMesh(core_axis_name="c", subcore_axis_name="s")
    f = pl.kernel(
        _transe_body,
        out_type=jax.ShapeDtypeStruct((BATCH,), jnp.float32),
        mesh=mesh,
        scratch_types=[
            pltpu.VMEM((_NCHUNK, _CHUNK), jnp.int32),
            pltpu.VMEM((_NCHUNK, _CHUNK), jnp.int32),
            pltpu.VMEM((_NCHUNK, _CHUNK), jnp.int32),
            pltpu.VMEM((_BPW, EMBED_DIM), jnp.float32),
            pltpu.VMEM((_BPW, EMBED_DIM), jnp.float32),
            pltpu.VMEM((_BPW, EMBED_DIM), jnp.float32),
            pltpu.VMEM((_BPW,), jnp.float32),
            pltpu.SemaphoreType.DMA,
        ],
        compiler_params=pltpu.CompilerParams(
            use_tc_tiling_on_sc=False, needs_layout_passes=False),
    )
    return f(heads, relations, tails, entity_emb, relation_emb)
